# baseline (speedup 1.0000x reference)
import functools

import jax
import jax.numpy as jnp
from jax import lax
from jax.experimental import pallas as pl
from jax.experimental.pallas import tpu as pltpu
from jax.experimental.pallas import tpu_sc as plsc

_D = 128
_B = 128
_K = 64
_NC = 2
_NS = 16
_NW = _NC * _NS
_BPW = _B // _NW
_NEG_PW = _BPW * _K


def _logsig(x):
  u = jnp.exp(-jnp.abs(x))
  z = u / (2.0 + u)
  z2 = z * z
  p = 1.0 + z2 * (1.0 / 3.0 + z2 * (1.0 / 5.0 + z2 * (
      1.0 / 7.0 + z2 * (1.0 / 9.0 + z2 * (1.0 / 11.0 + z2 * (1.0 / 13.0))))))
  return jnp.minimum(x, 0.0) - 2.0 * z * p


def _fused_sc(pos_input, pos_output, neg_idx, input_weight, output_weight):
  mesh = plsc.VectorSubcoreMesh(core_axis_name="c", subcore_axis_name="s")

  @functools.partial(
      pl.kernel,
      mesh=mesh,
      compiler_params=pltpu.CompilerParams(needs_layout_passes=False),
      out_type=jax.ShapeDtypeStruct((_NW, 16), jnp.float32),
      scratch_types=[
          pltpu.VMEM((8,), jnp.int32),
          pltpu.VMEM((16,), jnp.int32),
          pltpu.VMEM((2, 128), jnp.int32),
          pltpu.VMEM((8, _D), jnp.float32),
          pltpu.VMEM((16, _D), jnp.float32),
          pltpu.VMEM((_NEG_PW, _D), jnp.float32),
          pltpu.VMEM((16,), jnp.float32),
          pltpu.VMEM((8, 16), jnp.float32),
          pltpu.VMEM((16,), jnp.float32),
          pltpu.VMEM_SHARED((8, 16), jnp.float32),
          pltpu.SemaphoreType.DMA,
          pltpu.SemaphoreType.DMA,
          pltpu.SemaphoreType.DMA,
          pltpu.SemaphoreType.DMA,
      ],
  )
  def fused_kernel(pi_hbm, po_hbm, ineg_hbm, iw_hbm, ow_hbm, out_hbm,
                   pi8_v, po16_v, ineg_v, rin8_v, rout16_v, rneg_v,
                   rsv_v, rv_v, pv_v, rsum_shared, s0, s1, s2, s3):
    c = lax.axis_index("c")
    s = lax.axis_index("s")
    w = s * _NC + c
    par = w % 2
    iota = lax.iota(jnp.int32, 16)

    pltpu.sync_copy(pi_hbm.at[pl.ds(pl.multiple_of(8 * (w // 2), 8), 8)],
                    pi8_v)
    pltpu.sync_copy(ineg_hbm.at[w], ineg_v)

    cin = pltpu.async_copy(iw_hbm.at[pi8_v], rin8_v, s0)
    cn0 = pltpu.async_copy(ow_hbm.at[ineg_v.at[0]],
                           rneg_v.at[pl.ds(0, 128)], s1)
    cn1 = pltpu.async_copy(ow_hbm.at[ineg_v.at[1]],
                           rneg_v.at[pl.ds(128, 128)], s2)

    @pl.when(s < 8)
    def _rowsum():
      pltpu.sync_copy(po_hbm.at[pl.ds(pl.multiple_of(16 * s, 8), 16)],
                      po16_v)
      pltpu.async_copy(ow_hbm.at[po16_v], rout16_v, s3).wait()

      def rs_body(j, acc):
        g0 = plsc.load_gather(rout16_v, [iota, jnp.full((16,), 4 * j,
                                                        jnp.int32)])
        g1 = plsc.load_gather(rout16_v, [iota, jnp.full((16,), 4 * j + 1,
                                                        jnp.int32)])
        g2 = plsc.load_gather(rout16_v, [iota, jnp.full((16,), 4 * j + 2,
                                                        jnp.int32)])
        g3 = plsc.load_gather(rout16_v, [iota, jnp.full((16,), 4 * j + 3,
                                                        jnp.int32)])
        return acc + (g0 + g1) + (g2 + g3)

      acc = lax.fori_loop(0, _D // 4, rs_body, jnp.zeros((16,), jnp.float32))
      rsv_v[...] = acc
      pltpu.sync_copy(rsv_v, rsum_shared.at[s])

    cin.wait()
    cn0.wait()
    cn1.wait()
    rows = [(b * _K + kg * 16) + iota for b in range(_BPW) for kg in range(4)]

    def dot_body(jc, accs):
      base = jc * 16
      ebs = [rin8_v[par * _BPW + b, pl.ds(base, 16)] for b in range(_BPW)]
      new = list(accs)
      for dl in range(16):
        cols = jnp.full((16,), base + dl, jnp.int32)
        for b in range(_BPW):
          e = ebs[b][dl]
          for kg in range(4):
            t = b * 4 + kg
            g = plsc.load_gather(rneg_v, [rows[t], cols])
            new[t] = new[t] + g * e
      return tuple(new)

    accs = lax.fori_loop(
        0, _D // 16, dot_body,
        tuple(jnp.zeros((16,), jnp.float32) for _ in range(16)))

    plsc.subcore_barrier()
    pltpu.sync_copy(rsum_shared, rv_v)

    score_v = jnp.zeros((16,), jnp.float32)
    t_v = jnp.zeros((16,), jnp.float32)
    for b in range(_BPW):
      r0 = par * _BPW + b
      acc = rin8_v[r0, pl.ds(0, 16)] * rv_v[0]
      for j in range(1, 8):
        acc = acc + rin8_v[r0, pl.ds(16 * j, 16)] * rv_v[j]
      score_v = jnp.where(iota == b, jnp.sum(acc), score_v)
      sb = jnp.zeros((16,), jnp.float32)
      for kg in range(4):
        ns = jnp.clip(accs[b * 4 + kg], -10.0, 10.0)
        sb = sb + _logsig(-ns)
      t_v = jnp.where(iota == b, -jnp.sum(sb), t_v)

    pv = -_logsig(score_v) * (1.0 / _B) - _logsig(-t_v)
    pv_v[...] = jnp.where(iota < _BPW, pv, 0.0)
    pltpu.sync_copy(pv_v, out_hbm.at[w])

  return fused_kernel(pos_input, pos_output, neg_idx,
                      input_weight, output_weight)


def kernel(pos_input, pos_output, neg_v, input_weight, output_weight):
  pi = pos_input.astype(jnp.int32)
  po = pos_output.astype(jnp.int32)
  nv = neg_v.astype(jnp.int32).reshape(_NW, 2, 128)
  partials = _fused_sc(pi, po, nv, input_weight, output_weight)
  return jnp.sum(partials)
